# baseline (speedup 1.0000x reference)
import functools

import jax
import jax.numpy as jnp
from jax import lax
from jax.experimental import pallas as pl
from jax.experimental.pallas import tpu as pltpu
from jax.experimental.pallas import tpu_sc as plsc


def kernel(inputs, weight):
    *batches, d1, d2 = inputs.shape
    nbatch = 1
    for b in batches:
        nbatch *= b

    info = plsc.get_sparse_core_info()
    nworkers = info.num_cores * info.num_subcores
    rows_per_worker = d1 // nworkers

    mesh = plsc.VectorSubcoreMesh(core_axis_name="c", subcore_axis_name="s")

    @functools.partial(
        pl.kernel,
        mesh=mesh,
        out_type=jax.ShapeDtypeStruct((nbatch, d1, d2), weight.dtype),
        scratch_types=[pltpu.SemaphoreType.DMA],
    )
    def sc_copy(w_hbm, o_hbm, sem):
        wid = lax.axis_index("s") * info.num_cores + lax.axis_index("c")
        row0 = wid * rows_per_worker
        copies = [
            pltpu.make_async_copy(
                w_hbm.at[pl.ds(row0, rows_per_worker), :],
                o_hbm.at[b, pl.ds(row0, rows_per_worker), :],
                sem,
            )
            for b in range(nbatch)
        ]
        for cp in copies:
            cp.start()
        for cp in copies:
            cp.wait()

    out = sc_copy(weight)
    return out.reshape(tuple(batches) + (d1, d2))


# speedup vs baseline: 76.2365x; 76.2365x over previous
import jax
import jax.numpy as jnp
from jax.experimental import pallas as pl
from jax.experimental.pallas import tpu as pltpu


def kernel(inputs, weight):
    *batches, d1, d2 = inputs.shape
    nbatch = 1
    for b in batches:
        nbatch *= b

    block_rows = 2048
    nblocks = d1 // block_rows
    batch_pair = 2
    npairs = nbatch // batch_pair

    def body(w_ref, o_ref):
        o_ref[...] = jnp.broadcast_to(
            w_ref[...][None], (batch_pair, block_rows, d2)
        )

    out = pl.pallas_call(
        body,
        grid=(nblocks, npairs),
        in_specs=[pl.BlockSpec((block_rows, d2), lambda i, j: (i, 0))],
        out_specs=pl.BlockSpec(
            (batch_pair, block_rows, d2), lambda i, j: (j, i, 0)
        ),
        out_shape=jax.ShapeDtypeStruct((nbatch, d1, d2), weight.dtype),
    )(weight)

    return out.reshape(tuple(batches) + (d1, d2))
